# Initial kernel scaffold; baseline (speedup 1.0000x reference)
import jax
import jax.numpy as jnp
from jax.experimental import pallas as pl


def kernel(x, edge_attr, edge_index, Wn, bn, We, be):
    raise NotImplementedError("write your pallas kernel here")


# speedup vs baseline: 2.3948x; 2.3948x over previous
import functools

import jax
import jax.numpy as jnp
from jax import lax
from jax.experimental import pallas as pl
from jax.experimental.pallas import tpu as pltpu
from jax.experimental.pallas import tpu_sc as plsc

N_NODES = 10000
N_EDGES = 160000
D_NODE = 256
D_EDGE = 16
D_OUT = 256
HALF = 128
NC = 2
NS = 16
LANES = 16
K = 80
EPS = N_EDGES // NS
NCHUNK = EPS // K
ROWS_MAIN = 640
GR = 80


def _mm_body(a_ref, w_ref, b_ref, o_ref):
    acc = jnp.dot(a_ref[...], w_ref[...], preferred_element_type=jnp.float32)
    b = b_ref[pl.ds(pl.program_id(0), 1), :]
    o_ref[...] = jnp.maximum(acc + b, 0.0)


def _linear_relu_split(a, W, b2, rows, rb):
    nrb = rows // rb
    return pl.pallas_call(
        _mm_body,
        grid=(NC, nrb),
        in_specs=[
            pl.BlockSpec((rb, a.shape[1]), lambda c, r: (r, 0)),
            pl.BlockSpec((a.shape[1], HALF), lambda c, r: (0, c)),
            pl.BlockSpec((NC, HALF), lambda c, r: (0, 0)),
        ],
        out_specs=pl.BlockSpec((rb, HALF), lambda c, r, nrb=nrb: (c * nrb + r, 0)),
        out_shape=jax.ShapeDtypeStruct((NC * rows, HALF), jnp.float32),
    )(a, W, b2)


def _sc_body(w_hbm, h_hbm, srcr_hbm, dstr_hbm, out_hbm,
             src_b, dst_b, wrows, hrows, degv, ones_v,
             acc, dacc, gsem):
    c = lax.axis_index("c")
    s = lax.axis_index("s")
    r0 = s * ROWS_MAIN
    ngroups = jnp.where(s < NS - 1, ROWS_MAIN // GR,
                        (N_NODES - (NS - 1) * ROWS_MAIN) // GR)

    zero16 = jnp.zeros((LANES,), jnp.float32)
    one16 = jnp.ones((LANES,), jnp.float32)

    @pl.loop(0, GR)
    def _(k):
        for j in range(HALF // LANES):
            wrows[k, pl.ds(j * LANES, LANES)] = zero16

    @pl.loop(0, K // LANES)
    def _(q):
        ones_v[pl.ds(q * LANES, LANES)] = one16
        degv[pl.ds(q * LANES, LANES)] = zero16

    @pl.loop(0, ngroups)
    def _(g):
        rb = r0 + g * GR
        pltpu.sync_copy(wrows, acc.at[pl.ds(rb, GR)])
        pltpu.sync_copy(degv, dacc.at[pl.ds(rb, GR)])

    plsc.subcore_barrier()

    coff = c * N_NODES
    hbase = c * N_EDGES + s * EPS
    row0 = s * NCHUNK

    @pl.loop(0, NCHUNK)
    def _(i):
        pltpu.sync_copy(srcr_hbm.at[row0 + i], src_b)
        pltpu.sync_copy(dstr_hbm.at[row0 + i], dst_b)
        for j in range(K // LANES):
            sl = pl.ds(j * LANES, LANES)
            src_b[sl] = src_b[sl] + coff
        cp = pltpu.async_copy(w_hbm.at[src_b], wrows, gsem)
        pltpu.sync_copy(h_hbm.at[pl.ds(hbase + i * K, K)], hrows)
        cp.wait()

        @pl.loop(0, K)
        def _(k):
            for j in range(HALF // LANES):
                sl = (k, pl.ds(j * LANES, LANES))
                wrows[sl] = wrows[sl] * hrows[sl]

        pltpu.sync_copy(wrows, acc.at[dst_b], add=True)
        pltpu.sync_copy(ones_v, dacc.at[dst_b], add=True)

    plsc.subcore_barrier()

    @pl.loop(0, ngroups)
    def _(g):
        rb = r0 + g * GR
        pltpu.sync_copy(acc.at[pl.ds(rb, GR)], wrows)
        pltpu.sync_copy(dacc.at[pl.ds(rb, GR)], degv)

        @pl.loop(0, GR // LANES)
        def _(q):
            d = degv[pl.ds(q * LANES, LANES)]
            inv = 1.0 / jnp.maximum(d, 1.0)
            for kk in range(LANES):
                bc = jnp.take_along_axis(
                    inv, jnp.full((LANES,), kk, jnp.int32), axis=0
                )
                row = q * LANES + kk
                for j in range(HALF // LANES):
                    sl = (row, pl.ds(j * LANES, LANES))
                    wrows[sl] = wrows[sl] * bc

        pltpu.sync_copy(wrows, out_hbm.at[pl.ds(rb, GR), pl.ds(c * HALF, HALF)])


_sc_call = pl.kernel(
    _sc_body,
    out_type=jax.ShapeDtypeStruct((N_NODES, D_OUT), jnp.float32),
    mesh=plsc.VectorSubcoreMesh(
        core_axis_name="c", subcore_axis_name="s", num_cores=NC, num_subcores=NS
    ),
    compiler_params=pltpu.CompilerParams(use_tc_tiling_on_sc=False),
    scratch_types=[
        pltpu.VMEM((K,), jnp.int32),
        pltpu.VMEM((K,), jnp.int32),
        pltpu.VMEM((K, HALF), jnp.float32),
        pltpu.VMEM((K, HALF), jnp.float32),
        pltpu.VMEM((GR,), jnp.float32),
        pltpu.VMEM((K,), jnp.float32),
        pltpu.VMEM_SHARED((N_NODES, HALF), jnp.float32),
        pltpu.VMEM_SHARED((N_NODES,), jnp.float32),
        pltpu.SemaphoreType.DMA,
    ],
)


def kernel(x, edge_attr, edge_index, Wn, bn, We, be):
    ei = edge_index.astype(jnp.int32)
    srcr = ei[0].reshape(NS * NCHUNK, K)
    dstr = ei[1].reshape(NS * NCHUNK, K)
    w_cat = _linear_relu_split(x, Wn, bn.reshape(NC, HALF), N_NODES, 2000)
    h_cat = _linear_relu_split(edge_attr, We, be.reshape(NC, HALF), N_EDGES, 8000)
    return _sc_call(w_cat, h_cat, srcr, dstr)
